# Initial kernel scaffold; baseline (speedup 1.0000x reference)
import jax
import jax.numpy as jnp
from jax.experimental import pallas as pl


def kernel(x, edge_index, junc_index_pair, W0, b0, Wl1, bl1, Wr1, g1, be1, Wl2, bl2, Wr2, g2, be2, Wl3, bl3, Wr3, Wd, bd):
    raise NotImplementedError("write your pallas kernel here")


# speedup vs baseline: 2.8643x; 2.8643x over previous
import functools

import jax
import jax.numpy as jnp
from jax import lax
from jax.experimental import pallas as pl
from jax.experimental.pallas import tpu as pltpu
from jax.experimental.pallas import tpu_sc as plsc

N = 10000
E = 320000
NP = 100000
NC, NS = 2, 16
NW = NC * NS
NSEG = 320
NPAD = NW * NSEG
CCH = 4000
ECAP = E + CCH + 16
CG = 128
PSEG = 3136
PPAD = NW * PSEG


def _mesh():
    return plsc.VectorSubcoreMesh(core_axis_name="c", subcore_axis_name="s",
                                  num_cores=NC, num_subcores=NS)


_SC_PARAMS = pltpu.CompilerParams(use_tc_tiling_on_sc=False,
                                  needs_layout_passes=False)


def _wid():
    return lax.axis_index("s") * NC + lax.axis_index("c")


def _sc_compact(src, dst):
    nch = E // CCH

    @functools.partial(
        pl.kernel,
        mesh=_mesh(),
        compiler_params=_SC_PARAMS,
        out_type=(
            jax.ShapeDtypeStruct((NW, ECAP), jnp.int32),
            jax.ShapeDtypeStruct((NW, ECAP), jnp.int32),
            jax.ShapeDtypeStruct((NW, 16), jnp.int32),
        ),
        scratch_types=[
            pltpu.VMEM((CCH,), jnp.int32),
            pltpu.VMEM((CCH,), jnp.int32),
            pltpu.VMEM((CCH + 16,), jnp.int32),
            pltpu.VMEM((CCH + 16,), jnp.int32),
            pltpu.VMEM((16,), jnp.int32),
        ],
    )
    def k(src_hbm, dst_hbm, slist, dlist, counts, src_v, dst_v, sbuf, dbuf,
          cnt_v):
        w = _wid()
        lo = w * NSEG
        hi = lo + NSEG

        def chunk_body(ck, carry):
            ptr, total = carry
            base = pl.multiple_of(ck * CCH, CCH)
            pltpu.sync_copy(src_hbm.at[pl.ds(base, CCH)], src_v)
            pltpu.sync_copy(dst_hbm.at[pl.ds(base, CCH)], dst_v)

            def vec_body(j, ptr):
                off = pl.multiple_of(j * 16, 16)
                sv = src_v[pl.ds(off, 16)]
                dv = dst_v[pl.ds(off, 16)]
                m = (dv >= lo) & (dv < hi)
                cs = plsc.cumsum(m.astype(jnp.int32))
                pos = ptr + cs - 1
                plsc.store_scatter(sbuf, [pos], sv, mask=m)
                plsc.store_scatter(dbuf, [pos], dv - lo, mask=m)
                return ptr + cs[15]

            ptr = lax.fori_loop(0, CCH // 16, vec_body, ptr)
            f = ptr & ~15
            lv = sbuf[pl.ds(f, 16)]
            ld = dbuf[pl.ds(f, 16)]
            total = pl.multiple_of(total, 16)
            pltpu.sync_copy(sbuf.at[pl.ds(0, CCH)],
                            slist.at[w, pl.ds(total, CCH)])
            pltpu.sync_copy(dbuf.at[pl.ds(0, CCH)],
                            dlist.at[w, pl.ds(total, CCH)])
            sbuf[pl.ds(0, 16)] = lv
            dbuf[pl.ds(0, 16)] = ld
            return ptr - f, total + f

        ptr, total = lax.fori_loop(0, nch, chunk_body, (jnp.int32(0),
                                                        jnp.int32(0)))
        total = pl.multiple_of(total, 16)
        pltpu.sync_copy(sbuf.at[pl.ds(0, 16)], slist.at[w, pl.ds(total, 16)])
        pltpu.sync_copy(dbuf.at[pl.ds(0, 16)], dlist.at[w, pl.ds(total, 16)])
        cnt_v[...] = jnp.full((16,), total + ptr, jnp.int32)
        pltpu.sync_copy(cnt_v, counts.at[w])

    return k(src, dst)


def _sc_segmax(h, slist, dlist, counts, d):
    @functools.partial(
        pl.kernel,
        mesh=_mesh(),
        compiler_params=_SC_PARAMS,
        out_type=jax.ShapeDtypeStruct((NPAD, d), jnp.float32),
        scratch_types=[
            pltpu.VMEM((NSEG, d), jnp.float32),
            pltpu.VMEM((CG, d), jnp.float32),
            pltpu.VMEM((CG,), jnp.int32),
            pltpu.VMEM((CG + 16,), jnp.int32),
            pltpu.VMEM((16,), jnp.int32),
            pltpu.SemaphoreType.DMA,
        ],
    )
    def k(h_hbm, slist_hbm, dlist_hbm, counts_hbm, agg, acc, rows_v, idx_v,
          dl_v, cnt_v, sem):
        w = _wid()
        lo = w * NSEG
        pltpu.sync_copy(h_hbm.at[pl.ds(lo, NSEG)], acc)
        pltpu.sync_copy(counts_hbm.at[w], cnt_v)
        count = cnt_v[pl.ds(0, 16)][0]
        nch = lax.div(count + (CG - 1), CG)

        def chunk_body(ck, _):
            base = pl.multiple_of(ck * CG, CG)
            pltpu.sync_copy(slist_hbm.at[w, pl.ds(base, CG)], idx_v)
            pltpu.sync_copy(dlist_hbm.at[w, pl.ds(base, CG)],
                            dl_v.at[pl.ds(0, CG)])
            for j in range(CG // 16):
                v = idx_v[pl.ds(j * 16, 16)]
                idx_v[pl.ds(j * 16, 16)] = jnp.clip(v, 0, N - 1)
            pltpu.async_copy(h_hbm.at[idx_v], rows_v, sem).wait()
            m = jnp.minimum(CG, count - base)

            def edge_body(i, _):
                dl = dl_v[pl.ds(i, 16)][0]
                for cg in range(d // 16):
                    sl = pl.ds(cg * 16, 16)
                    acc[dl, sl] = jnp.maximum(acc[dl, sl], rows_v[i, sl])
                return 0

            lax.fori_loop(0, m, edge_body, 0)
            return 0

        lax.fori_loop(0, nch, chunk_body, 0)
        pltpu.sync_copy(acc, agg.at[pl.ds(lo, NSEG)])

    return k(h, slist, dlist, counts)


def _sc_decoder(p, q, sidx, eidx):
    @functools.partial(
        pl.kernel,
        mesh=_mesh(),
        compiler_params=_SC_PARAMS,
        out_type=jax.ShapeDtypeStruct((PPAD,), jnp.float32),
        scratch_types=[
            pltpu.VMEM((NPAD,), jnp.float32),
            pltpu.VMEM((NPAD,), jnp.float32),
            pltpu.VMEM((PSEG,), jnp.int32),
            pltpu.VMEM((PSEG,), jnp.int32),
            pltpu.VMEM((PSEG,), jnp.float32),
        ],
    )
    def k(p_hbm, q_hbm, s_hbm, e_hbm, out, p_v, q_v, s_v, e_v, o_v):
        w = _wid()
        base = w * PSEG
        pltpu.sync_copy(p_hbm, p_v)
        pltpu.sync_copy(q_hbm, q_v)
        pltpu.sync_copy(s_hbm.at[pl.ds(base, PSEG)], s_v)
        pltpu.sync_copy(e_hbm.at[pl.ds(base, PSEG)], e_v)

        def body(j, _):
            off = pl.multiple_of(j * 16, 16)
            sv = s_v[pl.ds(off, 16)]
            ev = e_v[pl.ds(off, 16)]
            pv = plsc.load_gather(p_v, [sv])
            qv = plsc.load_gather(q_v, [ev])
            z = pv + qv
            o_v[pl.ds(off, 16)] = 1.0 / (1.0 + jnp.exp(-z))
            return 0

        lax.fori_loop(0, PSEG // 16, body, 0)
        pltpu.sync_copy(o_v, out.at[pl.ds(base, PSEG)])

    return k(p, q, sidx, eidx)


def _tc_lin(x, W0, b0):
    def body(x_ref, w_ref, b_ref, o_ref):
        o_ref[...] = jnp.dot(x_ref[...], w_ref[...],
                             preferred_element_type=jnp.float32) + b_ref[...]

    return pl.pallas_call(
        body,
        out_shape=jax.ShapeDtypeStruct((NPAD, W0.shape[1]), jnp.float32),
    )(x, W0, b0)


def _tc_layer(agg, h, Wl, bl, Wr, g, be):
    dout = Wl.shape[1]

    def body(a_ref, h_ref, wl_ref, bl_ref, wr_ref, g_ref, be_ref, o_ref):
        z = (jnp.dot(a_ref[...], wl_ref[...],
                     preferred_element_type=jnp.float32)
             + jnp.dot(h_ref[...], wr_ref[...],
                       preferred_element_type=jnp.float32)
             + bl_ref[...])
        zv = z[:N]
        mu = jnp.mean(zv, axis=0, keepdims=True)
        var = jnp.mean((zv - mu) ** 2, axis=0, keepdims=True)
        o = (z - mu) * (g_ref[...] * lax.rsqrt(var + 1e-5)) + be_ref[...]
        o_ref[...] = jnp.maximum(o, 0.0)

    return pl.pallas_call(
        body,
        out_shape=jax.ShapeDtypeStruct((NPAD, dout), jnp.float32),
    )(agg, h, Wl, bl, Wr, g, be)


def _tc_final(agg, h, Wl, bl, Wr, W2, bd):
    def body(a_ref, h_ref, wl_ref, bl_ref, wr_ref, w2_ref, bd_ref, p_ref,
             q_ref):
        z = (jnp.dot(a_ref[...], wl_ref[...],
                     preferred_element_type=jnp.float32)
             + jnp.dot(h_ref[...], wr_ref[...],
                       preferred_element_type=jnp.float32)
             + bl_ref[...])
        pq = jnp.dot(z, w2_ref[...], preferred_element_type=jnp.float32)
        p_ref[...] = pq[:, 0:1] + bd_ref[...]
        q_ref[...] = pq[:, 1:2]

    return pl.pallas_call(
        body,
        out_shape=(
            jax.ShapeDtypeStruct((NPAD, 1), jnp.float32),
            jax.ShapeDtypeStruct((NPAD, 1), jnp.float32),
        ),
    )(agg, h, Wl, bl, Wr, W2, bd)


def kernel(x, edge_index, junc_index_pair, W0, b0, Wl1, bl1, Wr1, g1, be1,
           Wl2, bl2, Wr2, g2, be2, Wl3, bl3, Wr3, Wd, bd):
    src = edge_index[0]
    dst = edge_index[1]
    sidx = jnp.pad(junc_index_pair[:, 0], (0, PPAD - NP))
    eidx = jnp.pad(junc_index_pair[:, 1], (0, PPAD - NP))
    x_pad = jnp.pad(x, ((0, NPAD - N), (0, 0)))
    W2 = jnp.concatenate([Wd[:Wd.shape[0] // 2], Wd[Wd.shape[0] // 2:]],
                         axis=1)

    slist, dlist, counts = _sc_compact(src, dst)
    h0 = _tc_lin(x_pad, W0, b0.reshape(1, -1))
    agg1 = _sc_segmax(h0, slist, dlist, counts, h0.shape[1])
    h1 = _tc_layer(agg1, h0, Wl1, bl1.reshape(1, -1), Wr1, g1.reshape(1, -1),
                   be1.reshape(1, -1))
    agg2 = _sc_segmax(h1, slist, dlist, counts, h1.shape[1])
    h2 = _tc_layer(agg2, h1, Wl2, bl2.reshape(1, -1), Wr2, g2.reshape(1, -1),
                   be2.reshape(1, -1))
    agg3 = _sc_segmax(h2, slist, dlist, counts, h2.shape[1])
    p, q = _tc_final(agg3, h2, Wl3, bl3.reshape(1, -1), Wr3, W2,
                     bd.reshape(1, 1))
    out = _sc_decoder(p.reshape(NPAD), q.reshape(NPAD), sidx, eidx)
    return out[:NP]


# speedup vs baseline: 3.2161x; 1.1228x over previous
import functools

import jax
import jax.numpy as jnp
from jax import lax
from jax.experimental import pallas as pl
from jax.experimental.pallas import tpu as pltpu
from jax.experimental.pallas import tpu_sc as plsc

N = 10000
E = 320000
NP = 100000
NC, NS = 2, 16
NW = NC * NS
NSEG = 320
NPAD = NW * NSEG
CCH = 4000
ECAP = E + CCH + 16
CG = 128
PSEG = 3136
PPAD = NW * PSEG


def _mesh():
    return plsc.VectorSubcoreMesh(core_axis_name="c", subcore_axis_name="s",
                                  num_cores=NC, num_subcores=NS)


_SC_PARAMS = pltpu.CompilerParams(use_tc_tiling_on_sc=False,
                                  needs_layout_passes=False)


def _wid():
    return lax.axis_index("s") * NC + lax.axis_index("c")


def _sc_compact(src, dst):
    nch = E // CCH

    @functools.partial(
        pl.kernel,
        mesh=_mesh(),
        compiler_params=_SC_PARAMS,
        out_type=(
            jax.ShapeDtypeStruct((NW, ECAP), jnp.int32),
            jax.ShapeDtypeStruct((NW, ECAP), jnp.int32),
            jax.ShapeDtypeStruct((NW, 16), jnp.int32),
        ),
        scratch_types=[
            pltpu.VMEM((CCH,), jnp.int32),
            pltpu.VMEM((CCH,), jnp.int32),
            pltpu.VMEM((CCH + 16,), jnp.int32),
            pltpu.VMEM((CCH + 16,), jnp.int32),
            pltpu.VMEM((16,), jnp.int32),
        ],
    )
    def k(src_hbm, dst_hbm, slist, dlist, counts, src_v, dst_v, sbuf, dbuf,
          cnt_v):
        w = _wid()
        lo = w * NSEG
        hi = lo + NSEG

        def chunk_body(ck, carry):
            ptr, total = carry
            base = pl.multiple_of(ck * CCH, CCH)
            pltpu.sync_copy(src_hbm.at[pl.ds(base, CCH)], src_v)
            pltpu.sync_copy(dst_hbm.at[pl.ds(base, CCH)], dst_v)

            def vec_body(j, ptr):
                off = pl.multiple_of(j * 16, 16)
                sv = src_v[pl.ds(off, 16)]
                dv = dst_v[pl.ds(off, 16)]
                m = (dv >= lo) & (dv < hi)
                cs = plsc.cumsum(m.astype(jnp.int32))
                pos = ptr + cs - 1
                plsc.store_scatter(sbuf, [pos], sv, mask=m)
                plsc.store_scatter(dbuf, [pos], dv - lo, mask=m)
                return ptr + cs[15]

            ptr = lax.fori_loop(0, CCH // 16, vec_body, ptr)
            f = ptr & ~15
            lv = sbuf[pl.ds(f, 16)]
            ld = dbuf[pl.ds(f, 16)]
            total = pl.multiple_of(total, 16)
            pltpu.sync_copy(sbuf.at[pl.ds(0, CCH)],
                            slist.at[w, pl.ds(total, CCH)])
            pltpu.sync_copy(dbuf.at[pl.ds(0, CCH)],
                            dlist.at[w, pl.ds(total, CCH)])
            sbuf[pl.ds(0, 16)] = lv
            dbuf[pl.ds(0, 16)] = ld
            return ptr - f, total + f

        ptr, total = lax.fori_loop(0, nch, chunk_body, (jnp.int32(0),
                                                        jnp.int32(0)))
        pad = (16 - lax.rem(ptr, 16)) & 15
        mp = lax.broadcasted_iota(jnp.int32, (16,), 0) < pad
        csp = plsc.cumsum(mp.astype(jnp.int32))
        posp = ptr + csp - 1
        plsc.store_scatter(sbuf, [posp], jnp.full((16,), lo, jnp.int32),
                           mask=mp)
        plsc.store_scatter(dbuf, [posp], jnp.zeros((16,), jnp.int32), mask=mp)
        ptr = ptr + pad
        total = pl.multiple_of(total, 16)
        pltpu.sync_copy(sbuf.at[pl.ds(0, 16)], slist.at[w, pl.ds(total, 16)])
        pltpu.sync_copy(dbuf.at[pl.ds(0, 16)], dlist.at[w, pl.ds(total, 16)])
        cnt_v[...] = jnp.full((16,), total + ptr, jnp.int32)
        pltpu.sync_copy(cnt_v, counts.at[w])

    return k(src, dst)


def _sc_segmax(h, slist, dlist, counts, d):
    @functools.partial(
        pl.kernel,
        mesh=_mesh(),
        compiler_params=_SC_PARAMS,
        out_type=jax.ShapeDtypeStruct((NPAD, d), jnp.float32),
        scratch_types=[
            pltpu.VMEM((NSEG, d), jnp.float32),
            pltpu.VMEM((CG, d), jnp.float32),
            pltpu.VMEM((CG,), jnp.int32),
            pltpu.VMEM((CG + 16,), jnp.int32),
            pltpu.VMEM((16,), jnp.int32),
            pltpu.SemaphoreType.DMA,
        ],
    )
    def k(h_hbm, slist_hbm, dlist_hbm, counts_hbm, agg, acc, rows_v, idx_v,
          dl_v, cnt_v, sem):
        w = _wid()
        lo = w * NSEG
        pltpu.sync_copy(h_hbm.at[pl.ds(lo, NSEG)], acc)
        pltpu.sync_copy(counts_hbm.at[w], cnt_v)
        count = cnt_v[pl.ds(0, 16)][0]
        nch = lax.div(count + (CG - 1), CG)

        def chunk_body(ck, _):
            base = pl.multiple_of(ck * CG, CG)
            pltpu.sync_copy(slist_hbm.at[w, pl.ds(base, CG)], idx_v)
            pltpu.sync_copy(dlist_hbm.at[w, pl.ds(base, CG)],
                            dl_v.at[pl.ds(0, CG)])
            for j in range(CG // 16):
                v = idx_v[pl.ds(j * 16, 16)]
                idx_v[pl.ds(j * 16, 16)] = jnp.clip(v, 0, N - 1)
            pltpu.async_copy(h_hbm.at[idx_v], rows_v, sem).wait()
            m = jnp.minimum(CG, count - base)

            def group_body(g, _):
                goff = pl.multiple_of(g * 16, 16)
                dlv = dl_v[pl.ds(goff, 16)]
                for e in range(16):
                    dl = dlv[e]
                    for cg in range(d // 16):
                        sl = pl.ds(cg * 16, 16)
                        acc[dl, sl] = jnp.maximum(acc[dl, sl],
                                                  rows_v[goff + e, sl])
                return 0

            lax.fori_loop(0, lax.div(m, 16), group_body, 0)
            return 0

        lax.fori_loop(0, nch, chunk_body, 0)
        pltpu.sync_copy(acc, agg.at[pl.ds(lo, NSEG)])

    return k(h, slist, dlist, counts)


def _sc_decoder(p, q, sidx, eidx):
    @functools.partial(
        pl.kernel,
        mesh=_mesh(),
        compiler_params=_SC_PARAMS,
        out_type=jax.ShapeDtypeStruct((PPAD,), jnp.float32),
        scratch_types=[
            pltpu.VMEM((NPAD,), jnp.float32),
            pltpu.VMEM((NPAD,), jnp.float32),
            pltpu.VMEM((PSEG,), jnp.int32),
            pltpu.VMEM((PSEG,), jnp.int32),
            pltpu.VMEM((PSEG,), jnp.float32),
        ],
    )
    def k(p_hbm, q_hbm, s_hbm, e_hbm, out, p_v, q_v, s_v, e_v, o_v):
        w = _wid()
        base = w * PSEG
        pltpu.sync_copy(p_hbm, p_v)
        pltpu.sync_copy(q_hbm, q_v)
        pltpu.sync_copy(s_hbm.at[pl.ds(base, PSEG)], s_v)
        pltpu.sync_copy(e_hbm.at[pl.ds(base, PSEG)], e_v)

        def body(j, _):
            off = pl.multiple_of(j * 16, 16)
            sv = s_v[pl.ds(off, 16)]
            ev = e_v[pl.ds(off, 16)]
            pv = plsc.load_gather(p_v, [sv])
            qv = plsc.load_gather(q_v, [ev])
            z = pv + qv
            o_v[pl.ds(off, 16)] = 1.0 / (1.0 + jnp.exp(-z))
            return 0

        lax.fori_loop(0, PSEG // 16, body, 0)
        pltpu.sync_copy(o_v, out.at[pl.ds(base, PSEG)])

    return k(p, q, sidx, eidx)


def _tc_lin(x, W0, b0):
    def body(x_ref, w_ref, b_ref, o_ref):
        o_ref[...] = jnp.dot(x_ref[...], w_ref[...],
                             preferred_element_type=jnp.float32) + b_ref[...]

    return pl.pallas_call(
        body,
        out_shape=jax.ShapeDtypeStruct((NPAD, W0.shape[1]), jnp.float32),
    )(x, W0, b0)


def _tc_layer(agg, h, Wl, bl, Wr, g, be):
    dout = Wl.shape[1]

    def body(a_ref, h_ref, wl_ref, bl_ref, wr_ref, g_ref, be_ref, o_ref):
        z = (jnp.dot(a_ref[...], wl_ref[...],
                     preferred_element_type=jnp.float32)
             + jnp.dot(h_ref[...], wr_ref[...],
                       preferred_element_type=jnp.float32)
             + bl_ref[...])
        zv = z[:N]
        mu = jnp.mean(zv, axis=0, keepdims=True)
        var = jnp.mean((zv - mu) ** 2, axis=0, keepdims=True)
        o = (z - mu) * (g_ref[...] * lax.rsqrt(var + 1e-5)) + be_ref[...]
        o_ref[...] = jnp.maximum(o, 0.0)

    return pl.pallas_call(
        body,
        out_shape=jax.ShapeDtypeStruct((NPAD, dout), jnp.float32),
    )(agg, h, Wl, bl, Wr, g, be)


def _tc_final(agg, h, Wl, bl, Wr, W2, bd):
    def body(a_ref, h_ref, wl_ref, bl_ref, wr_ref, w2_ref, bd_ref, p_ref,
             q_ref):
        z = (jnp.dot(a_ref[...], wl_ref[...],
                     preferred_element_type=jnp.float32)
             + jnp.dot(h_ref[...], wr_ref[...],
                       preferred_element_type=jnp.float32)
             + bl_ref[...])
        pq = jnp.dot(z, w2_ref[...], preferred_element_type=jnp.float32)
        p_ref[...] = pq[:, 0:1] + bd_ref[...]
        q_ref[...] = pq[:, 1:2]

    return pl.pallas_call(
        body,
        out_shape=(
            jax.ShapeDtypeStruct((NPAD, 1), jnp.float32),
            jax.ShapeDtypeStruct((NPAD, 1), jnp.float32),
        ),
    )(agg, h, Wl, bl, Wr, W2, bd)


def kernel(x, edge_index, junc_index_pair, W0, b0, Wl1, bl1, Wr1, g1, be1,
           Wl2, bl2, Wr2, g2, be2, Wl3, bl3, Wr3, Wd, bd):
    src = edge_index[0]
    dst = edge_index[1]
    sidx = jnp.pad(junc_index_pair[:, 0], (0, PPAD - NP))
    eidx = jnp.pad(junc_index_pair[:, 1], (0, PPAD - NP))
    x_pad = jnp.pad(x, ((0, NPAD - N), (0, 0)))
    W2 = jnp.concatenate([Wd[:Wd.shape[0] // 2], Wd[Wd.shape[0] // 2:]],
                         axis=1)

    slist, dlist, counts = _sc_compact(src, dst)
    h0 = _tc_lin(x_pad, W0, b0.reshape(1, -1))
    agg1 = _sc_segmax(h0, slist, dlist, counts, h0.shape[1])
    h1 = _tc_layer(agg1, h0, Wl1, bl1.reshape(1, -1), Wr1, g1.reshape(1, -1),
                   be1.reshape(1, -1))
    agg2 = _sc_segmax(h1, slist, dlist, counts, h1.shape[1])
    h2 = _tc_layer(agg2, h1, Wl2, bl2.reshape(1, -1), Wr2, g2.reshape(1, -1),
                   be2.reshape(1, -1))
    agg3 = _sc_segmax(h2, slist, dlist, counts, h2.shape[1])
    p, q = _tc_final(agg3, h2, Wl3, bl3.reshape(1, -1), Wr3, W2,
                     bd.reshape(1, 1))
    out = _sc_decoder(p.reshape(NPAD), q.reshape(NPAD), sidx, eidx)
    return out[:NP]


# speedup vs baseline: 3.6866x; 1.1463x over previous
import functools

import jax
import jax.numpy as jnp
from jax import lax
from jax.experimental import pallas as pl
from jax.experimental.pallas import tpu as pltpu
from jax.experimental.pallas import tpu_sc as plsc

N = 10000
E = 320000
NP = 100000
NC, NS = 2, 16
NW = NC * NS
NSEG = 320
NPAD = NW * NSEG
CCH = 4000
ECAP = E + CCH + 16
CG = 128
PSEG = 3136
PPAD = NW * PSEG


def _mesh():
    return plsc.VectorSubcoreMesh(core_axis_name="c", subcore_axis_name="s",
                                  num_cores=NC, num_subcores=NS)


_SC_PARAMS = pltpu.CompilerParams(use_tc_tiling_on_sc=False,
                                  needs_layout_passes=False)


def _wid():
    return lax.axis_index("s") * NC + lax.axis_index("c")


def _sc_compact(src, dst):
    nch = E // CCH

    @functools.partial(
        pl.kernel,
        mesh=_mesh(),
        compiler_params=_SC_PARAMS,
        out_type=(
            jax.ShapeDtypeStruct((NW, ECAP), jnp.int32),
            jax.ShapeDtypeStruct((NW, ECAP), jnp.int32),
            jax.ShapeDtypeStruct((NW, 16), jnp.int32),
        ),
        scratch_types=[
            pltpu.VMEM((CCH,), jnp.int32),
            pltpu.VMEM((CCH,), jnp.int32),
            pltpu.VMEM((CCH + 16,), jnp.int32),
            pltpu.VMEM((CCH + 16,), jnp.int32),
            pltpu.VMEM((16,), jnp.int32),
        ],
    )
    def k(src_hbm, dst_hbm, slist, dlist, counts, src_v, dst_v, sbuf, dbuf,
          cnt_v):
        w = _wid()
        lo = w * NSEG
        hi = lo + NSEG

        def chunk_body(ck, carry):
            ptr, total = carry
            base = pl.multiple_of(ck * CCH, CCH)
            pltpu.sync_copy(src_hbm.at[pl.ds(base, CCH)], src_v)
            pltpu.sync_copy(dst_hbm.at[pl.ds(base, CCH)], dst_v)

            def vec_body(j, ptr):
                off = pl.multiple_of(j * 16, 16)
                sv = src_v[pl.ds(off, 16)]
                dv = dst_v[pl.ds(off, 16)]
                m = (dv >= lo) & (dv < hi)
                cs = plsc.cumsum(m.astype(jnp.int32))
                pos = ptr + cs - 1
                plsc.store_scatter(sbuf, [pos], sv, mask=m)
                plsc.store_scatter(dbuf, [pos], dv - lo, mask=m)
                return ptr + cs[15]

            ptr = lax.fori_loop(0, CCH // 16, vec_body, ptr)
            f = ptr & ~15
            lv = sbuf[pl.ds(f, 16)]
            ld = dbuf[pl.ds(f, 16)]
            total = pl.multiple_of(total, 16)
            pltpu.sync_copy(sbuf.at[pl.ds(0, CCH)],
                            slist.at[w, pl.ds(total, CCH)])
            pltpu.sync_copy(dbuf.at[pl.ds(0, CCH)],
                            dlist.at[w, pl.ds(total, CCH)])
            sbuf[pl.ds(0, 16)] = lv
            dbuf[pl.ds(0, 16)] = ld
            return ptr - f, total + f

        ptr, total = lax.fori_loop(0, nch, chunk_body, (jnp.int32(0),
                                                        jnp.int32(0)))
        pad = (16 - lax.rem(ptr, 16)) & 15
        mp = lax.broadcasted_iota(jnp.int32, (16,), 0) < pad
        csp = plsc.cumsum(mp.astype(jnp.int32))
        posp = ptr + csp - 1
        plsc.store_scatter(sbuf, [posp], jnp.full((16,), lo, jnp.int32),
                           mask=mp)
        plsc.store_scatter(dbuf, [posp], jnp.zeros((16,), jnp.int32), mask=mp)
        ptr = ptr + pad
        total = pl.multiple_of(total, 16)
        pltpu.sync_copy(sbuf.at[pl.ds(0, 16)], slist.at[w, pl.ds(total, 16)])
        pltpu.sync_copy(dbuf.at[pl.ds(0, 16)], dlist.at[w, pl.ds(total, 16)])
        cnt_v[...] = jnp.full((16,), total + ptr, jnp.int32)
        pltpu.sync_copy(cnt_v, counts.at[w])

    return k(src, dst)


def _sc_segmax(h, slist, dlist, counts, d):
    cg = 256 if d == 128 else 64

    @functools.partial(
        pl.kernel,
        mesh=_mesh(),
        compiler_params=_SC_PARAMS,
        out_type=jax.ShapeDtypeStruct((NPAD, d), jnp.float32),
        scratch_types=[
            pltpu.VMEM((NSEG, d), jnp.float32),
            pltpu.VMEM((cg, d), jnp.float32),
            pltpu.VMEM((cg, d), jnp.float32),
            pltpu.VMEM((cg,), jnp.int32),
            pltpu.VMEM((cg,), jnp.int32),
            pltpu.VMEM((cg,), jnp.int32),
            pltpu.VMEM((cg,), jnp.int32),
            pltpu.VMEM((16,), jnp.int32),
            pltpu.SemaphoreType.DMA,
            pltpu.SemaphoreType.DMA,
            pltpu.SemaphoreType.DMA,
            pltpu.SemaphoreType.DMA,
            pltpu.SemaphoreType.DMA,
            pltpu.SemaphoreType.DMA,
        ],
    )
    def k(h_hbm, slist_hbm, dlist_hbm, counts_hbm, agg, acc, rows0, rows1,
          idx0, idx1, dl0, dl1, cnt_v, ls0, ls1, ld0, ld1, g0, g1):
        rows = (rows0, rows1)
        idx = (idx0, idx1)
        dlb = (dl0, dl1)
        lssem = (ls0, ls1)
        ldsem = (ld0, ld1)
        gsem = (g0, g1)
        w = _wid()
        lo = w * NSEG
        pltpu.sync_copy(h_hbm.at[pl.ds(lo, NSEG)], acc)
        pltpu.sync_copy(counts_hbm.at[w], cnt_v)
        count = cnt_v[pl.ds(0, 16)][0]
        nch = lax.div(count + (cg - 1), cg)

        def lists_cp(ck, b):
            base = pl.multiple_of(ck * cg, cg)
            return (pltpu.make_async_copy(slist_hbm.at[w, pl.ds(base, cg)],
                                          idx[b], lssem[b]),
                    pltpu.make_async_copy(dlist_hbm.at[w, pl.ds(base, cg)],
                                          dlb[b], ldsem[b]))

        def gather_cp(b):
            return pltpu.make_async_copy(h_hbm.at[idx[b]], rows[b], gsem[b])

        def prep_gather(ck, b):
            a1, a2 = lists_cp(ck, b)
            a1.wait()
            a2.wait()
            for j in range(cg // 16):
                v = idx[b][pl.ds(j * 16, 16)]
                idx[b][pl.ds(j * 16, 16)] = jnp.clip(v, 0, N - 1)
            gather_cp(b).start()

        @pl.when(nch > 0)
        def _prologue():
            a1, a2 = lists_cp(0, 0)
            a1.start()
            a2.start()

            @pl.when(nch > 1)
            def _():
                b1, b2 = lists_cp(1, 1)
                b1.start()
                b2.start()

            prep_gather(0, 0)

        def pair_body(kk, _):
            for b in range(2):
                ck = kk * 2 + b

                @pl.when(ck < nch)
                def _(ck=ck, b=b):
                    nb = 1 - b
                    gather_cp(b).wait()

                    @pl.when(ck + 1 < nch)
                    def _():
                        prep_gather(ck + 1, nb)

                    base = pl.multiple_of(ck * cg, cg)
                    m = jnp.minimum(cg, count - base)

                    def group_body(g, _):
                        goff = pl.multiple_of(g * 16, 16)
                        dlv = dlb[b][pl.ds(goff, 16)]
                        for e in range(16):
                            dl = dlv[e]
                            for c in range(d // 16):
                                sl = pl.ds(c * 16, 16)
                                acc[dl, sl] = jnp.maximum(
                                    acc[dl, sl], rows[b][goff + e, sl])
                        return 0

                    lax.fori_loop(0, lax.div(m, 16), group_body, 0)

                    @pl.when(ck + 2 < nch)
                    def _():
                        a1, a2 = lists_cp(ck + 2, b)
                        a1.start()
                        a2.start()

            return 0

        lax.fori_loop(0, lax.div(nch + 1, 2), pair_body, 0)
        pltpu.sync_copy(acc, agg.at[pl.ds(lo, NSEG)])

    return k(h, slist, dlist, counts)


def _sc_decoder(p, q, sidx, eidx):
    @functools.partial(
        pl.kernel,
        mesh=_mesh(),
        compiler_params=_SC_PARAMS,
        out_type=jax.ShapeDtypeStruct((PPAD,), jnp.float32),
        scratch_types=[
            pltpu.VMEM((NPAD,), jnp.float32),
            pltpu.VMEM((NPAD,), jnp.float32),
            pltpu.VMEM((PSEG,), jnp.int32),
            pltpu.VMEM((PSEG,), jnp.int32),
            pltpu.VMEM((PSEG,), jnp.float32),
        ],
    )
    def k(p_hbm, q_hbm, s_hbm, e_hbm, out, p_v, q_v, s_v, e_v, o_v):
        w = _wid()
        base = w * PSEG
        pltpu.sync_copy(p_hbm, p_v)
        pltpu.sync_copy(q_hbm, q_v)
        pltpu.sync_copy(s_hbm.at[pl.ds(base, PSEG)], s_v)
        pltpu.sync_copy(e_hbm.at[pl.ds(base, PSEG)], e_v)

        def body(j, _):
            off = pl.multiple_of(j * 16, 16)
            sv = s_v[pl.ds(off, 16)]
            ev = e_v[pl.ds(off, 16)]
            pv = plsc.load_gather(p_v, [sv])
            qv = plsc.load_gather(q_v, [ev])
            z = pv + qv
            o_v[pl.ds(off, 16)] = 1.0 / (1.0 + jnp.exp(-z))
            return 0

        lax.fori_loop(0, PSEG // 16, body, 0)
        pltpu.sync_copy(o_v, out.at[pl.ds(base, PSEG)])

    return k(p, q, sidx, eidx)


def _tc_lin(x, W0, b0):
    def body(x_ref, w_ref, b_ref, o_ref):
        o_ref[...] = jnp.dot(x_ref[...], w_ref[...],
                             preferred_element_type=jnp.float32) + b_ref[...]

    return pl.pallas_call(
        body,
        out_shape=jax.ShapeDtypeStruct((NPAD, W0.shape[1]), jnp.float32),
    )(x, W0, b0)


def _tc_layer(agg, h, Wl, bl, Wr, g, be):
    dout = Wl.shape[1]

    def body(a_ref, h_ref, wl_ref, bl_ref, wr_ref, g_ref, be_ref, o_ref):
        z = (jnp.dot(a_ref[...], wl_ref[...],
                     preferred_element_type=jnp.float32)
             + jnp.dot(h_ref[...], wr_ref[...],
                       preferred_element_type=jnp.float32)
             + bl_ref[...])
        zv = z[:N]
        mu = jnp.mean(zv, axis=0, keepdims=True)
        var = jnp.mean((zv - mu) ** 2, axis=0, keepdims=True)
        o = (z - mu) * (g_ref[...] * lax.rsqrt(var + 1e-5)) + be_ref[...]
        o_ref[...] = jnp.maximum(o, 0.0)

    return pl.pallas_call(
        body,
        out_shape=jax.ShapeDtypeStruct((NPAD, dout), jnp.float32),
    )(agg, h, Wl, bl, Wr, g, be)


def _tc_final(agg, h, Wl, bl, Wr, W2, bd):
    def body(a_ref, h_ref, wl_ref, bl_ref, wr_ref, w2_ref, bd_ref, p_ref,
             q_ref):
        z = (jnp.dot(a_ref[...], wl_ref[...],
                     preferred_element_type=jnp.float32)
             + jnp.dot(h_ref[...], wr_ref[...],
                       preferred_element_type=jnp.float32)
             + bl_ref[...])
        pq = jnp.dot(z, w2_ref[...], preferred_element_type=jnp.float32)
        p_ref[...] = pq[:, 0:1] + bd_ref[...]
        q_ref[...] = pq[:, 1:2]

    return pl.pallas_call(
        body,
        out_shape=(
            jax.ShapeDtypeStruct((NPAD, 1), jnp.float32),
            jax.ShapeDtypeStruct((NPAD, 1), jnp.float32),
        ),
    )(agg, h, Wl, bl, Wr, W2, bd)


def kernel(x, edge_index, junc_index_pair, W0, b0, Wl1, bl1, Wr1, g1, be1,
           Wl2, bl2, Wr2, g2, be2, Wl3, bl3, Wr3, Wd, bd):
    src = edge_index[0]
    dst = edge_index[1]
    sidx = jnp.pad(junc_index_pair[:, 0], (0, PPAD - NP))
    eidx = jnp.pad(junc_index_pair[:, 1], (0, PPAD - NP))
    x_pad = jnp.pad(x, ((0, NPAD - N), (0, 0)))
    W2 = jnp.concatenate([Wd[:Wd.shape[0] // 2], Wd[Wd.shape[0] // 2:]],
                         axis=1)

    slist, dlist, counts = _sc_compact(src, dst)
    h0 = _tc_lin(x_pad, W0, b0.reshape(1, -1))
    agg1 = _sc_segmax(h0, slist, dlist, counts, h0.shape[1])
    h1 = _tc_layer(agg1, h0, Wl1, bl1.reshape(1, -1), Wr1, g1.reshape(1, -1),
                   be1.reshape(1, -1))
    agg2 = _sc_segmax(h1, slist, dlist, counts, h1.shape[1])
    h2 = _tc_layer(agg2, h1, Wl2, bl2.reshape(1, -1), Wr2, g2.reshape(1, -1),
                   be2.reshape(1, -1))
    agg3 = _sc_segmax(h2, slist, dlist, counts, h2.shape[1])
    p, q = _tc_final(agg3, h2, Wl3, bl3.reshape(1, -1), Wr3, W2,
                     bd.reshape(1, 1))
    out = _sc_decoder(p.reshape(NPAD), q.reshape(NPAD), sidx, eidx)
    return out[:NP]


# speedup vs baseline: 5.1258x; 1.3904x over previous
import functools

import jax
import jax.numpy as jnp
from jax import lax
from jax.experimental import pallas as pl
from jax.experimental.pallas import tpu as pltpu
from jax.experimental.pallas import tpu_sc as plsc

N = 10000
E = 320000
NP = 100000
NC, NS = 2, 16
NW = NC * NS
NSEG = 320
NPAD = NW * NSEG
CCH = 4000
ECAP = E + CCH + 16
CG = 128
VCAP = 32768
PSEG = 3136
PPAD = NW * PSEG


def _mesh():
    return plsc.VectorSubcoreMesh(core_axis_name="c", subcore_axis_name="s",
                                  num_cores=NC, num_subcores=NS)


_SC_PARAMS = pltpu.CompilerParams(use_tc_tiling_on_sc=False,
                                  needs_layout_passes=False)


def _wid():
    return lax.axis_index("s") * NC + lax.axis_index("c")


def _sc_compact(src, dst):
    nch = E // CCH

    @functools.partial(
        pl.kernel,
        mesh=_mesh(),
        compiler_params=_SC_PARAMS,
        out_type=(
            jax.ShapeDtypeStruct((NW, ECAP), jnp.int32),
            jax.ShapeDtypeStruct((NW, ECAP), jnp.int32),
            jax.ShapeDtypeStruct((NW, 16), jnp.int32),
        ),
        scratch_types=[
            pltpu.VMEM((CCH,), jnp.int32),
            pltpu.VMEM((CCH,), jnp.int32),
            pltpu.VMEM((CCH + 16,), jnp.int32),
            pltpu.VMEM((CCH + 16,), jnp.int32),
            pltpu.VMEM((16,), jnp.int32),
            pltpu.VMEM((NSEG,), jnp.int32),
            pltpu.VMEM((NSEG,), jnp.int32),
            pltpu.VMEM((VCAP + CCH,), jnp.int32),
            pltpu.VMEM((VCAP + CCH,), jnp.int32),
        ],
    )
    def k(src_hbm, dst_hbm, slist, dlist, counts, src_v, dst_v, sbuf, dbuf,
          cnt_v, bins, fill, sorted_s, sorted_d):
        w = _wid()
        lo = w * NSEG
        hi = lo + NSEG

        def chunk_body(ck, carry):
            ptr, total = carry
            base = pl.multiple_of(ck * CCH, CCH)
            pltpu.sync_copy(src_hbm.at[pl.ds(base, CCH)], src_v)
            pltpu.sync_copy(dst_hbm.at[pl.ds(base, CCH)], dst_v)

            def vec_body(j, ptr):
                off = pl.multiple_of(j * 16, 16)
                sv = src_v[pl.ds(off, 16)]
                dv = dst_v[pl.ds(off, 16)]
                m = (dv >= lo) & (dv < hi)
                cs = plsc.cumsum(m.astype(jnp.int32))
                pos = ptr + cs - 1
                plsc.store_scatter(sbuf, [pos], sv, mask=m)
                plsc.store_scatter(dbuf, [pos], dv - lo, mask=m)
                return ptr + cs[15]

            ptr = lax.fori_loop(0, CCH // 16, vec_body, ptr)
            f = ptr & ~15
            lv = sbuf[pl.ds(f, 16)]
            ld = dbuf[pl.ds(f, 16)]
            total = pl.multiple_of(total, 16)
            pltpu.sync_copy(sbuf.at[pl.ds(0, CCH)],
                            slist.at[w, pl.ds(total, CCH)])
            pltpu.sync_copy(dbuf.at[pl.ds(0, CCH)],
                            dlist.at[w, pl.ds(total, CCH)])
            sbuf[pl.ds(0, 16)] = lv
            dbuf[pl.ds(0, 16)] = ld
            return ptr - f, total + f

        ptr, total = lax.fori_loop(0, nch, chunk_body, (jnp.int32(0),
                                                        jnp.int32(0)))
        pad = (16 - lax.rem(ptr, 16)) & 15
        mp = lax.broadcasted_iota(jnp.int32, (16,), 0) < pad
        csp = plsc.cumsum(mp.astype(jnp.int32))
        posp = ptr + csp - 1
        plsc.store_scatter(sbuf, [posp], jnp.full((16,), lo, jnp.int32),
                           mask=mp)
        plsc.store_scatter(dbuf, [posp], jnp.zeros((16,), jnp.int32), mask=mp)
        ptr = ptr + pad
        total = pl.multiple_of(total, 16)
        pltpu.sync_copy(sbuf.at[pl.ds(0, 16)], slist.at[w, pl.ds(total, 16)])
        pltpu.sync_copy(dbuf.at[pl.ds(0, 16)], dlist.at[w, pl.ds(total, 16)])
        count = total + ptr

        in_vmem = count <= VCAP
        ii = lax.broadcasted_iota(jnp.int32, (16,), 0)

        def seg_info(sk):
            prev = sk.at[jnp.maximum(ii - 1, 0)].get(mode="promise_in_bounds")
            nxt = sk.at[jnp.minimum(ii + 1, 15)].get(mode="promise_in_bounds")
            is_start = (ii == 0) | (sk != prev)
            is_last = (ii == 15) | (sk != nxt)
            mstart = plsc.cummax(jnp.where(is_start, ii, 0))
            return is_last, mstart

        @pl.when(in_vmem & (count > 0))
        def _sort():
            for j in range(NSEG // 16):
                bins[pl.ds(j * 16, 16)] = jnp.zeros((16,), jnp.int32)
            nck = lax.div(count + (CCH - 1), CCH)

            def hist_chunk(ck, _):
                base = pl.multiple_of(ck * CCH, CCH)
                pltpu.sync_copy(dlist.at[w, pl.ds(base, CCH)], dst_v)
                nv = lax.div(jnp.minimum(CCH, count - base), 16)

                def hist_vec(j, _):
                    off = pl.multiple_of(j * 16, 16)
                    sk = plsc.sort_key_val(dst_v[pl.ds(off, 16)], ii)[0]
                    is_last, mstart = seg_info(sk)
                    plsc.addupdate_scatter(bins, [sk], ii - mstart + 1,
                                           mask=is_last)
                    return 0

                lax.fori_loop(0, nv, hist_vec, 0)
                return 0

            lax.fori_loop(0, nck, hist_chunk, 0)

            def prefix(j, carry):
                off = pl.multiple_of(j * 16, 16)
                b16 = bins[pl.ds(off, 16)]
                cs = plsc.cumsum(b16)
                fill[pl.ds(off, 16)] = carry + cs - b16
                return carry + cs[15]

            lax.fori_loop(0, NSEG // 16, prefix, jnp.int32(0))

            def perm_chunk(ck, _):
                base = pl.multiple_of(ck * CCH, CCH)
                pltpu.sync_copy(slist.at[w, pl.ds(base, CCH)], src_v)
                pltpu.sync_copy(dlist.at[w, pl.ds(base, CCH)], dst_v)
                nv = lax.div(jnp.minimum(CCH, count - base), 16)

                def perm_vec(j, _):
                    off = pl.multiple_of(j * 16, 16)
                    sk, perm = plsc.sort_key_val(dst_v[pl.ds(off, 16)], ii)
                    svp = src_v[pl.ds(off, 16)].at[perm].get(mode="promise_in_bounds")
                    is_last, mstart = seg_info(sk)
                    basef = plsc.load_gather(fill, [sk])
                    pos = basef + (ii - mstart)
                    plsc.store_scatter(sorted_s, [pos], svp)
                    plsc.store_scatter(sorted_d, [pos], sk)
                    plsc.addupdate_scatter(fill, [sk], ii - mstart + 1,
                                           mask=is_last)
                    return 0

                lax.fori_loop(0, nv, perm_vec, 0)
                return 0

            lax.fori_loop(0, nck, perm_chunk, 0)

            def flush_chunk(ck, _):
                base = pl.multiple_of(ck * CCH, CCH)
                pltpu.sync_copy(sorted_s.at[pl.ds(base, CCH)],
                                slist.at[w, pl.ds(base, CCH)])
                pltpu.sync_copy(sorted_d.at[pl.ds(base, CCH)],
                                dlist.at[w, pl.ds(base, CCH)])
                return 0

            lax.fori_loop(0, nck, flush_chunk, 0)

        flag = jnp.where(in_vmem, 1, 0)
        cnt_vec = jnp.full((16,), count, jnp.int32)
        cnt_v[...] = jnp.where(ii == 1, flag, cnt_vec)
        pltpu.sync_copy(cnt_v, counts.at[w])

    return k(src, dst)


def _sc_segmax(h, slist, dlist, counts, d):
    cg = 256 if d == 128 else 64

    @functools.partial(
        pl.kernel,
        mesh=_mesh(),
        compiler_params=_SC_PARAMS,
        out_type=jax.ShapeDtypeStruct((NPAD, d), jnp.float32),
        scratch_types=[
            pltpu.VMEM((NSEG, d), jnp.float32),
            pltpu.VMEM((cg, d), jnp.float32),
            pltpu.VMEM((cg, d), jnp.float32),
            pltpu.VMEM((cg,), jnp.int32),
            pltpu.VMEM((cg,), jnp.int32),
            pltpu.VMEM((cg,), jnp.int32),
            pltpu.VMEM((cg,), jnp.int32),
            pltpu.VMEM((16,), jnp.int32),
            pltpu.SemaphoreType.DMA,
            pltpu.SemaphoreType.DMA,
            pltpu.SemaphoreType.DMA,
            pltpu.SemaphoreType.DMA,
            pltpu.SemaphoreType.DMA,
            pltpu.SemaphoreType.DMA,
        ],
    )
    def k(h_hbm, slist_hbm, dlist_hbm, counts_hbm, agg, acc, rows0, rows1,
          idx0, idx1, dl0, dl1, cnt_v, ls0, ls1, ld0, ld1, g0, g1):
        rows = (rows0, rows1)
        idx = (idx0, idx1)
        dlb = (dl0, dl1)
        lssem = (ls0, ls1)
        ldsem = (ld0, ld1)
        gsem = (g0, g1)
        w = _wid()
        lo = w * NSEG
        pltpu.sync_copy(h_hbm.at[pl.ds(lo, NSEG)], acc)
        pltpu.sync_copy(counts_hbm.at[w], cnt_v)
        cvec = cnt_v[pl.ds(0, 16)]
        count = cvec[0]
        flag = cvec[1]
        nch = lax.div(count + (cg - 1), cg)
        R = d // 16

        def lists_cp(ck, b):
            base = pl.multiple_of(ck * cg, cg)
            return (pltpu.make_async_copy(slist_hbm.at[w, pl.ds(base, cg)],
                                          idx[b], lssem[b]),
                    pltpu.make_async_copy(dlist_hbm.at[w, pl.ds(base, cg)],
                                          dlb[b], ldsem[b]))

        def gather_cp(b):
            return pltpu.make_async_copy(h_hbm.at[idx[b]], rows[b], gsem[b])

        def prep_gather(ck, b):
            a1, a2 = lists_cp(ck, b)
            a1.wait()
            a2.wait()
            for j in range(cg // 16):
                v = idx[b][pl.ds(j * 16, 16)]
                idx[b][pl.ds(j * 16, 16)] = jnp.clip(v, 0, N - 1)
            gather_cp(b).start()

        @pl.when(nch > 0)
        def _prologue():
            a1, a2 = lists_cp(0, 0)
            a1.start()
            a2.start()

            @pl.when(nch > 1)
            def _():
                b1, b2 = lists_cp(1, 1)
                b1.start()
                b2.start()

            prep_gather(0, 0)

        def do_chunk(ck, b, carry):
            nb = 1 - b
            gather_cp(b).wait()

            @pl.when(ck + 1 < nch)
            def _():
                prep_gather(ck + 1, nb)

            base = pl.multiple_of(ck * cg, cg)
            m = jnp.minimum(cg, count - base)
            ng = lax.div(m, 16)

            def group_sorted(g, c):
                goff = pl.multiple_of(g * 16, 16)
                dlv = dlb[b][pl.ds(goff, 16)]
                cur = c[0]
                regs = list(c[1:])
                for e in range(16):
                    dl = dlv[e]

                    def switch(args, dl=dl):
                        oc = args[0]
                        for ci in range(R):
                            acc[oc, pl.ds(ci * 16, 16)] = args[1 + ci]
                        return (dl,) + tuple(acc[dl, pl.ds(ci * 16, 16)]
                                             for ci in range(R))

                    out = lax.cond(dl != cur, switch, lambda a: a,
                                   (cur, *regs))
                    cur = out[0]
                    regs = list(out[1:])
                    for ci in range(R):
                        sl = pl.ds(ci * 16, 16)
                        regs[ci] = jnp.maximum(regs[ci],
                                               rows[b][goff + e, sl])
                return (cur, *regs)

            def group_unsorted(g, c):
                goff = pl.multiple_of(g * 16, 16)
                dlv = dlb[b][pl.ds(goff, 16)]
                for e in range(16):
                    dl = dlv[e]
                    for ci in range(R):
                        sl = pl.ds(ci * 16, 16)
                        acc[dl, sl] = jnp.maximum(acc[dl, sl],
                                                  rows[b][goff + e, sl])
                return c

            carry = lax.cond(
                flag == 1,
                lambda c: lax.fori_loop(0, ng, group_sorted, c),
                lambda c: lax.fori_loop(0, ng, group_unsorted, c),
                carry)

            @pl.when(ck + 2 < nch)
            def _():
                a1, a2 = lists_cp(ck + 2, b)
                a1.start()
                a2.start()

            return carry

        def pair_body(kk, carry):
            for b in range(2):
                ck = kk * 2 + b
                carry = lax.cond(
                    ck < nch,
                    lambda c, ck=ck, b=b: do_chunk(ck, b, c),
                    lambda c: c, carry)
            return carry

        init = (jnp.int32(0),) + tuple(acc[0, pl.ds(ci * 16, 16)]
                                       for ci in range(R))
        fin = lax.fori_loop(0, lax.div(nch + 1, 2), pair_body, init)

        @pl.when(flag == 1)
        def _flush():
            for ci in range(R):
                acc[fin[0], pl.ds(ci * 16, 16)] = fin[1 + ci]

        pltpu.sync_copy(acc, agg.at[pl.ds(lo, NSEG)])

    return k(h, slist, dlist, counts)


def _sc_decoder(p, q, sidx, eidx):
    @functools.partial(
        pl.kernel,
        mesh=_mesh(),
        compiler_params=_SC_PARAMS,
        out_type=jax.ShapeDtypeStruct((PPAD,), jnp.float32),
        scratch_types=[
            pltpu.VMEM((NPAD,), jnp.float32),
            pltpu.VMEM((NPAD,), jnp.float32),
            pltpu.VMEM((PSEG,), jnp.int32),
            pltpu.VMEM((PSEG,), jnp.int32),
            pltpu.VMEM((PSEG,), jnp.float32),
        ],
    )
    def k(p_hbm, q_hbm, s_hbm, e_hbm, out, p_v, q_v, s_v, e_v, o_v):
        w = _wid()
        base = w * PSEG
        pltpu.sync_copy(p_hbm, p_v)
        pltpu.sync_copy(q_hbm, q_v)
        pltpu.sync_copy(s_hbm.at[pl.ds(base, PSEG)], s_v)
        pltpu.sync_copy(e_hbm.at[pl.ds(base, PSEG)], e_v)

        def body(j, _):
            off = pl.multiple_of(j * 16, 16)
            sv = s_v[pl.ds(off, 16)]
            ev = e_v[pl.ds(off, 16)]
            pv = plsc.load_gather(p_v, [sv])
            qv = plsc.load_gather(q_v, [ev])
            z = pv + qv
            o_v[pl.ds(off, 16)] = 1.0 / (1.0 + jnp.exp(-z))
            return 0

        lax.fori_loop(0, PSEG // 16, body, 0)
        pltpu.sync_copy(o_v, out.at[pl.ds(base, PSEG)])

    return k(p, q, sidx, eidx)


def _tc_lin(x, W0, b0):
    def body(x_ref, w_ref, b_ref, o_ref):
        o_ref[...] = jnp.dot(x_ref[...], w_ref[...],
                             preferred_element_type=jnp.float32) + b_ref[...]

    return pl.pallas_call(
        body,
        out_shape=jax.ShapeDtypeStruct((NPAD, W0.shape[1]), jnp.float32),
    )(x, W0, b0)


def _tc_layer(agg, h, Wl, bl, Wr, g, be):
    dout = Wl.shape[1]

    def body(a_ref, h_ref, wl_ref, bl_ref, wr_ref, g_ref, be_ref, o_ref):
        z = (jnp.dot(a_ref[...], wl_ref[...],
                     preferred_element_type=jnp.float32)
             + jnp.dot(h_ref[...], wr_ref[...],
                       preferred_element_type=jnp.float32)
             + bl_ref[...])
        zv = z[:N]
        mu = jnp.mean(zv, axis=0, keepdims=True)
        var = jnp.mean((zv - mu) ** 2, axis=0, keepdims=True)
        o = (z - mu) * (g_ref[...] * lax.rsqrt(var + 1e-5)) + be_ref[...]
        o_ref[...] = jnp.maximum(o, 0.0)

    return pl.pallas_call(
        body,
        out_shape=jax.ShapeDtypeStruct((NPAD, dout), jnp.float32),
    )(agg, h, Wl, bl, Wr, g, be)


def _tc_final(agg, h, Wl, bl, Wr, W2, bd):
    def body(a_ref, h_ref, wl_ref, bl_ref, wr_ref, w2_ref, bd_ref, p_ref,
             q_ref):
        z = (jnp.dot(a_ref[...], wl_ref[...],
                     preferred_element_type=jnp.float32)
             + jnp.dot(h_ref[...], wr_ref[...],
                       preferred_element_type=jnp.float32)
             + bl_ref[...])
        pq = jnp.dot(z, w2_ref[...], preferred_element_type=jnp.float32)
        p_ref[...] = pq[:, 0:1] + bd_ref[...]
        q_ref[...] = pq[:, 1:2]

    return pl.pallas_call(
        body,
        out_shape=(
            jax.ShapeDtypeStruct((NPAD, 1), jnp.float32),
            jax.ShapeDtypeStruct((NPAD, 1), jnp.float32),
        ),
    )(agg, h, Wl, bl, Wr, W2, bd)


def kernel(x, edge_index, junc_index_pair, W0, b0, Wl1, bl1, Wr1, g1, be1,
           Wl2, bl2, Wr2, g2, be2, Wl3, bl3, Wr3, Wd, bd):
    src = edge_index[0]
    dst = edge_index[1]
    sidx = jnp.pad(junc_index_pair[:, 0], (0, PPAD - NP))
    eidx = jnp.pad(junc_index_pair[:, 1], (0, PPAD - NP))
    x_pad = jnp.pad(x, ((0, NPAD - N), (0, 0)))
    W2 = jnp.concatenate([Wd[:Wd.shape[0] // 2], Wd[Wd.shape[0] // 2:]],
                         axis=1)

    slist, dlist, counts = _sc_compact(src, dst)
    h0 = _tc_lin(x_pad, W0, b0.reshape(1, -1))
    agg1 = _sc_segmax(h0, slist, dlist, counts, h0.shape[1])
    h1 = _tc_layer(agg1, h0, Wl1, bl1.reshape(1, -1), Wr1, g1.reshape(1, -1),
                   be1.reshape(1, -1))
    agg2 = _sc_segmax(h1, slist, dlist, counts, h1.shape[1])
    h2 = _tc_layer(agg2, h1, Wl2, bl2.reshape(1, -1), Wr2, g2.reshape(1, -1),
                   be2.reshape(1, -1))
    agg3 = _sc_segmax(h2, slist, dlist, counts, h2.shape[1])
    p, q = _tc_final(agg3, h2, Wl3, bl3.reshape(1, -1), Wr3, W2,
                     bd.reshape(1, 1))
    out = _sc_decoder(p.reshape(NPAD), q.reshape(NPAD), sidx, eidx)
    return out[:NP]


# speedup vs baseline: 5.1758x; 1.0098x over previous
import functools

import jax
import jax.numpy as jnp
from jax import lax
from jax.experimental import pallas as pl
from jax.experimental.pallas import tpu as pltpu
from jax.experimental.pallas import tpu_sc as plsc

N = 10000
E = 320000
NP = 100000
NC, NS = 2, 16
NW = NC * NS
NSEG = 320
NPAD = NW * NSEG
CCH = 4000
ECAP = E + CCH + 16
CG = 128
VCAP = 32768
PSEG = 3136
PPAD = NW * PSEG


def _mesh():
    return plsc.VectorSubcoreMesh(core_axis_name="c", subcore_axis_name="s",
                                  num_cores=NC, num_subcores=NS)


_SC_PARAMS = pltpu.CompilerParams(use_tc_tiling_on_sc=False,
                                  needs_layout_passes=False)


def _wid():
    return lax.axis_index("s") * NC + lax.axis_index("c")


def _sc_compact(src, dst):
    nch = E // CCH

    @functools.partial(
        pl.kernel,
        mesh=_mesh(),
        compiler_params=_SC_PARAMS,
        out_type=(
            jax.ShapeDtypeStruct((NW, ECAP), jnp.int32),
            jax.ShapeDtypeStruct((NW, ECAP), jnp.int32),
            jax.ShapeDtypeStruct((NW, 16), jnp.int32),
        ),
        scratch_types=[
            pltpu.VMEM((CCH,), jnp.int32),
            pltpu.VMEM((CCH,), jnp.int32),
            pltpu.VMEM((CCH + 16,), jnp.int32),
            pltpu.VMEM((CCH + 16,), jnp.int32),
            pltpu.VMEM((16,), jnp.int32),
            pltpu.VMEM((NSEG,), jnp.int32),
            pltpu.VMEM((NSEG,), jnp.int32),
            pltpu.VMEM((VCAP + CCH,), jnp.int32),
            pltpu.VMEM((VCAP + CCH,), jnp.int32),
        ],
    )
    def k(src_hbm, dst_hbm, slist, dlist, counts, src_v, dst_v, sbuf, dbuf,
          cnt_v, bins, fill, sorted_s, sorted_d):
        w = _wid()
        lo = w * NSEG
        hi = lo + NSEG

        def chunk_body(ck, carry):
            ptr, total = carry
            base = pl.multiple_of(ck * CCH, CCH)
            pltpu.sync_copy(src_hbm.at[pl.ds(base, CCH)], src_v)
            pltpu.sync_copy(dst_hbm.at[pl.ds(base, CCH)], dst_v)

            def vec_body(j, ptr):
                off = pl.multiple_of(j * 16, 16)
                sv = src_v[pl.ds(off, 16)]
                dv = dst_v[pl.ds(off, 16)]
                m = (dv >= lo) & (dv < hi)
                cs = plsc.cumsum(m.astype(jnp.int32))
                pos = ptr + cs - 1
                plsc.store_scatter(sbuf, [pos], sv, mask=m)
                plsc.store_scatter(dbuf, [pos], dv - lo, mask=m)
                return ptr + plsc.all_reduce_population_count(m)[0]

            ptr = lax.fori_loop(0, CCH // 16, vec_body, ptr)
            f = ptr & ~15
            lv = sbuf[pl.ds(f, 16)]
            ld = dbuf[pl.ds(f, 16)]
            total = pl.multiple_of(total, 16)
            pltpu.sync_copy(sbuf.at[pl.ds(0, CCH)],
                            slist.at[w, pl.ds(total, CCH)])
            pltpu.sync_copy(dbuf.at[pl.ds(0, CCH)],
                            dlist.at[w, pl.ds(total, CCH)])
            sbuf[pl.ds(0, 16)] = lv
            dbuf[pl.ds(0, 16)] = ld
            return ptr - f, total + f

        ptr, total = lax.fori_loop(0, nch, chunk_body, (jnp.int32(0),
                                                        jnp.int32(0)))
        pad = (16 - lax.rem(ptr, 16)) & 15
        mp = lax.broadcasted_iota(jnp.int32, (16,), 0) < pad
        csp = plsc.cumsum(mp.astype(jnp.int32))
        posp = ptr + csp - 1
        plsc.store_scatter(sbuf, [posp], jnp.full((16,), lo, jnp.int32),
                           mask=mp)
        plsc.store_scatter(dbuf, [posp], jnp.zeros((16,), jnp.int32), mask=mp)
        ptr = ptr + pad
        total = pl.multiple_of(total, 16)
        pltpu.sync_copy(sbuf.at[pl.ds(0, 16)], slist.at[w, pl.ds(total, 16)])
        pltpu.sync_copy(dbuf.at[pl.ds(0, 16)], dlist.at[w, pl.ds(total, 16)])
        count = total + ptr

        in_vmem = count <= VCAP
        ii = lax.broadcasted_iota(jnp.int32, (16,), 0)

        def seg_info(sk):
            prev = sk.at[jnp.maximum(ii - 1, 0)].get(mode="promise_in_bounds")
            nxt = sk.at[jnp.minimum(ii + 1, 15)].get(mode="promise_in_bounds")
            is_start = (ii == 0) | (sk != prev)
            is_last = (ii == 15) | (sk != nxt)
            mstart = plsc.cummax(jnp.where(is_start, ii, 0))
            return is_last, mstart

        @pl.when(in_vmem & (count > 0))
        def _sort():
            for j in range(NSEG // 16):
                bins[pl.ds(j * 16, 16)] = jnp.zeros((16,), jnp.int32)
            nck = lax.div(count + (CCH - 1), CCH)

            def hist_chunk(ck, _):
                base = pl.multiple_of(ck * CCH, CCH)
                pltpu.sync_copy(dlist.at[w, pl.ds(base, CCH)], dst_v)
                nv = lax.div(jnp.minimum(CCH, count - base), 16)

                def hist_vec(j, _):
                    off = pl.multiple_of(j * 16, 16)
                    sk = plsc.sort_key_val(dst_v[pl.ds(off, 16)], ii)[0]
                    is_last, mstart = seg_info(sk)
                    plsc.addupdate_scatter(bins, [sk], ii - mstart + 1,
                                           mask=is_last)
                    return 0

                lax.fori_loop(0, nv, hist_vec, 0)
                return 0

            lax.fori_loop(0, nck, hist_chunk, 0)

            def prefix(j, carry):
                off = pl.multiple_of(j * 16, 16)
                b16 = bins[pl.ds(off, 16)]
                cs = plsc.cumsum(b16)
                fill[pl.ds(off, 16)] = carry + cs - b16
                return carry + cs[15]

            lax.fori_loop(0, NSEG // 16, prefix, jnp.int32(0))

            def perm_chunk(ck, _):
                base = pl.multiple_of(ck * CCH, CCH)
                pltpu.sync_copy(slist.at[w, pl.ds(base, CCH)], src_v)
                pltpu.sync_copy(dlist.at[w, pl.ds(base, CCH)], dst_v)
                nv = lax.div(jnp.minimum(CCH, count - base), 16)

                def perm_vec(j, _):
                    off = pl.multiple_of(j * 16, 16)
                    sk, perm = plsc.sort_key_val(dst_v[pl.ds(off, 16)], ii)
                    svp = src_v[pl.ds(off, 16)].at[perm].get(mode="promise_in_bounds")
                    is_last, mstart = seg_info(sk)
                    basef = plsc.load_gather(fill, [sk])
                    pos = basef + (ii - mstart)
                    plsc.store_scatter(sorted_s, [pos], svp)
                    plsc.store_scatter(sorted_d, [pos], sk)
                    plsc.addupdate_scatter(fill, [sk], ii - mstart + 1,
                                           mask=is_last)
                    return 0

                lax.fori_loop(0, nv, perm_vec, 0)
                return 0

            lax.fori_loop(0, nck, perm_chunk, 0)

            def flush_chunk(ck, _):
                base = pl.multiple_of(ck * CCH, CCH)
                pltpu.sync_copy(sorted_s.at[pl.ds(base, CCH)],
                                slist.at[w, pl.ds(base, CCH)])
                pltpu.sync_copy(sorted_d.at[pl.ds(base, CCH)],
                                dlist.at[w, pl.ds(base, CCH)])
                return 0

            lax.fori_loop(0, nck, flush_chunk, 0)

        flag = jnp.where(in_vmem, 1, 0)
        cnt_vec = jnp.full((16,), count, jnp.int32)
        cnt_v[...] = jnp.where(ii == 1, flag, cnt_vec)
        pltpu.sync_copy(cnt_v, counts.at[w])

    return k(src, dst)


def _sc_segmax(h, slist, dlist, counts, d):
    cg = 320 if d == 128 else 80

    @functools.partial(
        pl.kernel,
        mesh=_mesh(),
        compiler_params=_SC_PARAMS,
        out_type=jax.ShapeDtypeStruct((NPAD, d), jnp.float32),
        scratch_types=[
            pltpu.VMEM((NSEG, d), jnp.float32),
            pltpu.VMEM((cg, d), jnp.float32),
            pltpu.VMEM((cg, d), jnp.float32),
            pltpu.VMEM((cg,), jnp.int32),
            pltpu.VMEM((cg,), jnp.int32),
            pltpu.VMEM((cg,), jnp.int32),
            pltpu.VMEM((cg,), jnp.int32),
            pltpu.VMEM((16,), jnp.int32),
            pltpu.SemaphoreType.DMA,
            pltpu.SemaphoreType.DMA,
            pltpu.SemaphoreType.DMA,
            pltpu.SemaphoreType.DMA,
            pltpu.SemaphoreType.DMA,
            pltpu.SemaphoreType.DMA,
        ],
    )
    def k(h_hbm, slist_hbm, dlist_hbm, counts_hbm, agg, acc, rows0, rows1,
          idx0, idx1, dl0, dl1, cnt_v, ls0, ls1, ld0, ld1, g0, g1):
        rows = (rows0, rows1)
        idx = (idx0, idx1)
        dlb = (dl0, dl1)
        lssem = (ls0, ls1)
        ldsem = (ld0, ld1)
        gsem = (g0, g1)
        w = _wid()
        lo = w * NSEG
        pltpu.sync_copy(h_hbm.at[pl.ds(lo, NSEG)], acc)
        pltpu.sync_copy(counts_hbm.at[w], cnt_v)
        cvec = cnt_v[pl.ds(0, 16)]
        count = cvec[0]
        flag = cvec[1]
        nch = lax.div(count + (cg - 1), cg)
        R = d // 16

        def lists_cp(ck, b):
            base = pl.multiple_of(ck * cg, cg)
            return (pltpu.make_async_copy(slist_hbm.at[w, pl.ds(base, cg)],
                                          idx[b], lssem[b]),
                    pltpu.make_async_copy(dlist_hbm.at[w, pl.ds(base, cg)],
                                          dlb[b], ldsem[b]))

        def gather_cp(b):
            return pltpu.make_async_copy(h_hbm.at[idx[b]], rows[b], gsem[b])

        def prep_gather(ck, b):
            a1, a2 = lists_cp(ck, b)
            a1.wait()
            a2.wait()
            for j in range(cg // 16):
                v = idx[b][pl.ds(j * 16, 16)]
                idx[b][pl.ds(j * 16, 16)] = jnp.clip(v, 0, N - 1)
            gather_cp(b).start()

        @pl.when(nch > 0)
        def _prologue():
            a1, a2 = lists_cp(0, 0)
            a1.start()
            a2.start()

            @pl.when(nch > 1)
            def _():
                b1, b2 = lists_cp(1, 1)
                b1.start()
                b2.start()

            prep_gather(0, 0)

        def do_chunk(ck, b, carry):
            nb = 1 - b
            gather_cp(b).wait()

            @pl.when(ck + 1 < nch)
            def _():
                prep_gather(ck + 1, nb)

            base = pl.multiple_of(ck * cg, cg)
            m = jnp.minimum(cg, count - base)
            ng = lax.div(m, 16)

            def group_sorted(g, c):
                goff = pl.multiple_of(g * 16, 16)
                dlv = dlb[b][pl.ds(goff, 16)]
                cur = c[0]
                regs = list(c[1:])
                for e in range(16):
                    dl = dlv[e]

                    def switch(args, dl=dl):
                        oc = args[0]
                        for ci in range(R):
                            acc[oc, pl.ds(ci * 16, 16)] = args[1 + ci]
                        return (dl,) + tuple(acc[dl, pl.ds(ci * 16, 16)]
                                             for ci in range(R))

                    out = lax.cond(dl != cur, switch, lambda a: a,
                                   (cur, *regs))
                    cur = out[0]
                    regs = list(out[1:])
                    for ci in range(R):
                        sl = pl.ds(ci * 16, 16)
                        regs[ci] = jnp.maximum(regs[ci],
                                               rows[b][goff + e, sl])
                return (cur, *regs)

            def group_unsorted(g, c):
                goff = pl.multiple_of(g * 16, 16)
                dlv = dlb[b][pl.ds(goff, 16)]
                for e in range(16):
                    dl = dlv[e]
                    for ci in range(R):
                        sl = pl.ds(ci * 16, 16)
                        acc[dl, sl] = jnp.maximum(acc[dl, sl],
                                                  rows[b][goff + e, sl])
                return c

            carry = lax.cond(
                flag == 1,
                lambda c: lax.fori_loop(0, ng, group_sorted, c),
                lambda c: lax.fori_loop(0, ng, group_unsorted, c),
                carry)

            @pl.when(ck + 2 < nch)
            def _():
                a1, a2 = lists_cp(ck + 2, b)
                a1.start()
                a2.start()

            return carry

        def pair_body(kk, carry):
            for b in range(2):
                ck = kk * 2 + b
                carry = lax.cond(
                    ck < nch,
                    lambda c, ck=ck, b=b: do_chunk(ck, b, c),
                    lambda c: c, carry)
            return carry

        init = (jnp.int32(0),) + tuple(acc[0, pl.ds(ci * 16, 16)]
                                       for ci in range(R))
        fin = lax.fori_loop(0, lax.div(nch + 1, 2), pair_body, init)

        @pl.when(flag == 1)
        def _flush():
            for ci in range(R):
                acc[fin[0], pl.ds(ci * 16, 16)] = fin[1 + ci]

        pltpu.sync_copy(acc, agg.at[pl.ds(lo, NSEG)])

    return k(h, slist, dlist, counts)


def _sc_decoder(p, q, sidx, eidx):
    @functools.partial(
        pl.kernel,
        mesh=_mesh(),
        compiler_params=_SC_PARAMS,
        out_type=jax.ShapeDtypeStruct((PPAD,), jnp.float32),
        scratch_types=[
            pltpu.VMEM((NPAD,), jnp.float32),
            pltpu.VMEM((NPAD,), jnp.float32),
            pltpu.VMEM((PSEG,), jnp.int32),
            pltpu.VMEM((PSEG,), jnp.int32),
            pltpu.VMEM((PSEG,), jnp.float32),
        ],
    )
    def k(p_hbm, q_hbm, s_hbm, e_hbm, out, p_v, q_v, s_v, e_v, o_v):
        w = _wid()
        base = w * PSEG
        pltpu.sync_copy(p_hbm, p_v)
        pltpu.sync_copy(q_hbm, q_v)
        pltpu.sync_copy(s_hbm.at[pl.ds(base, PSEG)], s_v)
        pltpu.sync_copy(e_hbm.at[pl.ds(base, PSEG)], e_v)

        def body(j, _):
            off = pl.multiple_of(j * 16, 16)
            sv = s_v[pl.ds(off, 16)]
            ev = e_v[pl.ds(off, 16)]
            pv = plsc.load_gather(p_v, [sv])
            qv = plsc.load_gather(q_v, [ev])
            z = pv + qv
            o_v[pl.ds(off, 16)] = 1.0 / (1.0 + jnp.exp(-z))
            return 0

        lax.fori_loop(0, PSEG // 16, body, 0)
        pltpu.sync_copy(o_v, out.at[pl.ds(base, PSEG)])

    return k(p, q, sidx, eidx)


def _tc_lin(x, W0, b0):
    def body(x_ref, w_ref, b_ref, o_ref):
        o_ref[...] = jnp.dot(x_ref[...], w_ref[...],
                             preferred_element_type=jnp.float32) + b_ref[...]

    return pl.pallas_call(
        body,
        out_shape=jax.ShapeDtypeStruct((NPAD, W0.shape[1]), jnp.float32),
    )(x, W0, b0)


def _tc_layer(agg, h, Wl, bl, Wr, g, be):
    dout = Wl.shape[1]

    def body(a_ref, h_ref, wl_ref, bl_ref, wr_ref, g_ref, be_ref, o_ref):
        z = (jnp.dot(a_ref[...], wl_ref[...],
                     preferred_element_type=jnp.float32)
             + jnp.dot(h_ref[...], wr_ref[...],
                       preferred_element_type=jnp.float32)
             + bl_ref[...])
        zv = z[:N]
        mu = jnp.mean(zv, axis=0, keepdims=True)
        var = jnp.mean((zv - mu) ** 2, axis=0, keepdims=True)
        o = (z - mu) * (g_ref[...] * lax.rsqrt(var + 1e-5)) + be_ref[...]
        o_ref[...] = jnp.maximum(o, 0.0)

    return pl.pallas_call(
        body,
        out_shape=jax.ShapeDtypeStruct((NPAD, dout), jnp.float32),
    )(agg, h, Wl, bl, Wr, g, be)


def _tc_final(agg, h, Wl, bl, Wr, W2, bd):
    def body(a_ref, h_ref, wl_ref, bl_ref, wr_ref, w2_ref, bd_ref, p_ref,
             q_ref):
        z = (jnp.dot(a_ref[...], wl_ref[...],
                     preferred_element_type=jnp.float32)
             + jnp.dot(h_ref[...], wr_ref[...],
                       preferred_element_type=jnp.float32)
             + bl_ref[...])
        pq = jnp.dot(z, w2_ref[...], preferred_element_type=jnp.float32)
        p_ref[...] = pq[:, 0:1] + bd_ref[...]
        q_ref[...] = pq[:, 1:2]

    return pl.pallas_call(
        body,
        out_shape=(
            jax.ShapeDtypeStruct((NPAD, 1), jnp.float32),
            jax.ShapeDtypeStruct((NPAD, 1), jnp.float32),
        ),
    )(agg, h, Wl, bl, Wr, W2, bd)


def kernel(x, edge_index, junc_index_pair, W0, b0, Wl1, bl1, Wr1, g1, be1,
           Wl2, bl2, Wr2, g2, be2, Wl3, bl3, Wr3, Wd, bd):
    src = edge_index[0]
    dst = edge_index[1]
    sidx = jnp.pad(junc_index_pair[:, 0], (0, PPAD - NP))
    eidx = jnp.pad(junc_index_pair[:, 1], (0, PPAD - NP))
    x_pad = jnp.pad(x, ((0, NPAD - N), (0, 0)))
    W2 = jnp.concatenate([Wd[:Wd.shape[0] // 2], Wd[Wd.shape[0] // 2:]],
                         axis=1)

    slist, dlist, counts = _sc_compact(src, dst)
    h0 = _tc_lin(x_pad, W0, b0.reshape(1, -1))
    agg1 = _sc_segmax(h0, slist, dlist, counts, h0.shape[1])
    h1 = _tc_layer(agg1, h0, Wl1, bl1.reshape(1, -1), Wr1, g1.reshape(1, -1),
                   be1.reshape(1, -1))
    agg2 = _sc_segmax(h1, slist, dlist, counts, h1.shape[1])
    h2 = _tc_layer(agg2, h1, Wl2, bl2.reshape(1, -1), Wr2, g2.reshape(1, -1),
                   be2.reshape(1, -1))
    agg3 = _sc_segmax(h2, slist, dlist, counts, h2.shape[1])
    p, q = _tc_final(agg3, h2, Wl3, bl3.reshape(1, -1), Wr3, W2,
                     bd.reshape(1, 1))
    out = _sc_decoder(p.reshape(NPAD), q.reshape(NPAD), sidx, eidx)
    return out[:NP]


# speedup vs baseline: 5.3059x; 1.0251x over previous
import functools

import jax
import jax.numpy as jnp
from jax import lax
from jax.experimental import pallas as pl
from jax.experimental.pallas import tpu as pltpu
from jax.experimental.pallas import tpu_sc as plsc

N = 10000
E = 320000
NP = 100000
NC, NS = 2, 16
NW = NC * NS
NSEG = 320
NPAD = NW * NSEG
CCH = 8000
ECAP = E + CCH + 16
CG = 128
VCAP = 32768
PSEG = 3136
PPAD = NW * PSEG


def _mesh():
    return plsc.VectorSubcoreMesh(core_axis_name="c", subcore_axis_name="s",
                                  num_cores=NC, num_subcores=NS)


_SC_PARAMS = pltpu.CompilerParams(use_tc_tiling_on_sc=False,
                                  needs_layout_passes=False)


def _wid():
    return lax.axis_index("s") * NC + lax.axis_index("c")


def _sc_compact(src, dst):
    nch = E // CCH

    @functools.partial(
        pl.kernel,
        mesh=_mesh(),
        compiler_params=_SC_PARAMS,
        out_type=(
            jax.ShapeDtypeStruct((NW, ECAP), jnp.int32),
            jax.ShapeDtypeStruct((NW, ECAP), jnp.int32),
            jax.ShapeDtypeStruct((NW, 16), jnp.int32),
        ),
        scratch_types=[
            pltpu.VMEM((CCH,), jnp.int32),
            pltpu.VMEM((CCH,), jnp.int32),
            pltpu.VMEM((CCH + 16,), jnp.int32),
            pltpu.VMEM((CCH + 16,), jnp.int32),
            pltpu.VMEM((16,), jnp.int32),
            pltpu.VMEM((NSEG,), jnp.int32),
            pltpu.VMEM((NSEG,), jnp.int32),
            pltpu.VMEM((VCAP + CCH,), jnp.int32),
            pltpu.VMEM((VCAP + CCH,), jnp.int32),
        ],
    )
    def k(src_hbm, dst_hbm, slist, dlist, counts, src_v, dst_v, sbuf, dbuf,
          cnt_v, bins, fill, sorted_s, sorted_d):
        w = _wid()
        lo = w * NSEG
        hi = lo + NSEG

        def chunk_body(ck, carry):
            ptr, total = carry
            base = pl.multiple_of(ck * CCH, CCH)
            pltpu.sync_copy(src_hbm.at[pl.ds(base, CCH)], src_v)
            pltpu.sync_copy(dst_hbm.at[pl.ds(base, CCH)], dst_v)

            def vec_body(j, ptr):
                off = pl.multiple_of(j * 16, 16)
                sv = src_v[pl.ds(off, 16)]
                dv = dst_v[pl.ds(off, 16)]
                m = (dv >= lo) & (dv < hi)
                cs = plsc.cumsum(m.astype(jnp.int32))
                pos = ptr + cs - 1
                plsc.store_scatter(sbuf, [pos], sv, mask=m)
                plsc.store_scatter(dbuf, [pos], dv - lo, mask=m)
                return ptr + plsc.all_reduce_population_count(m)[0]

            ptr = lax.fori_loop(0, CCH // 16, vec_body, ptr)
            f = ptr & ~15
            lv = sbuf[pl.ds(f, 16)]
            ld = dbuf[pl.ds(f, 16)]
            total = pl.multiple_of(total, 16)
            pltpu.sync_copy(sbuf.at[pl.ds(0, CCH)],
                            slist.at[w, pl.ds(total, CCH)])
            pltpu.sync_copy(dbuf.at[pl.ds(0, CCH)],
                            dlist.at[w, pl.ds(total, CCH)])
            sbuf[pl.ds(0, 16)] = lv
            dbuf[pl.ds(0, 16)] = ld
            return ptr - f, total + f

        ptr, total = lax.fori_loop(0, nch, chunk_body, (jnp.int32(0),
                                                        jnp.int32(0)))
        pad = (16 - lax.rem(ptr, 16)) & 15
        mp = lax.broadcasted_iota(jnp.int32, (16,), 0) < pad
        csp = plsc.cumsum(mp.astype(jnp.int32))
        posp = ptr + csp - 1
        plsc.store_scatter(sbuf, [posp], jnp.full((16,), lo, jnp.int32),
                           mask=mp)
        plsc.store_scatter(dbuf, [posp], jnp.zeros((16,), jnp.int32), mask=mp)
        ptr = ptr + pad
        total = pl.multiple_of(total, 16)
        pltpu.sync_copy(sbuf.at[pl.ds(0, 16)], slist.at[w, pl.ds(total, 16)])
        pltpu.sync_copy(dbuf.at[pl.ds(0, 16)], dlist.at[w, pl.ds(total, 16)])
        count = total + ptr

        in_vmem = count <= VCAP
        ii = lax.broadcasted_iota(jnp.int32, (16,), 0)

        def seg_info(sk):
            prev = sk.at[jnp.maximum(ii - 1, 0)].get(mode="promise_in_bounds")
            nxt = sk.at[jnp.minimum(ii + 1, 15)].get(mode="promise_in_bounds")
            is_start = (ii == 0) | (sk != prev)
            is_last = (ii == 15) | (sk != nxt)
            mstart = plsc.cummax(jnp.where(is_start, ii, 0))
            return is_last, mstart

        @pl.when(in_vmem & (count > 0))
        def _sort():
            for j in range(NSEG // 16):
                bins[pl.ds(j * 16, 16)] = jnp.zeros((16,), jnp.int32)
            nck = lax.div(count + (CCH - 1), CCH)

            def hist_chunk(ck, _):
                base = pl.multiple_of(ck * CCH, CCH)
                pltpu.sync_copy(dlist.at[w, pl.ds(base, CCH)], dst_v)
                nv = lax.div(jnp.minimum(CCH, count - base), 16)

                def hist_vec(j, _):
                    off = pl.multiple_of(j * 16, 16)
                    sk = plsc.sort_key_val(dst_v[pl.ds(off, 16)], ii)[0]
                    is_last, mstart = seg_info(sk)
                    plsc.addupdate_scatter(bins, [sk], ii - mstart + 1,
                                           mask=is_last)
                    return 0

                lax.fori_loop(0, nv, hist_vec, 0)
                return 0

            lax.fori_loop(0, nck, hist_chunk, 0)

            def prefix(j, carry):
                off = pl.multiple_of(j * 16, 16)
                b16 = bins[pl.ds(off, 16)]
                cs = plsc.cumsum(b16)
                fill[pl.ds(off, 16)] = carry + cs - b16
                return carry + cs[15]

            lax.fori_loop(0, NSEG // 16, prefix, jnp.int32(0))

            def perm_chunk(ck, _):
                base = pl.multiple_of(ck * CCH, CCH)
                pltpu.sync_copy(slist.at[w, pl.ds(base, CCH)], src_v)
                pltpu.sync_copy(dlist.at[w, pl.ds(base, CCH)], dst_v)
                nv = lax.div(jnp.minimum(CCH, count - base), 16)

                def perm_vec(j, _):
                    off = pl.multiple_of(j * 16, 16)
                    sk, perm = plsc.sort_key_val(dst_v[pl.ds(off, 16)], ii)
                    svp = src_v[pl.ds(off, 16)].at[perm].get(mode="promise_in_bounds")
                    is_last, mstart = seg_info(sk)
                    basef = plsc.load_gather(fill, [sk])
                    pos = basef + (ii - mstart)
                    plsc.store_scatter(sorted_s, [pos], svp)
                    plsc.store_scatter(sorted_d, [pos], sk)
                    plsc.addupdate_scatter(fill, [sk], ii - mstart + 1,
                                           mask=is_last)
                    return 0

                lax.fori_loop(0, nv, perm_vec, 0)
                return 0

            lax.fori_loop(0, nck, perm_chunk, 0)

            def flush_chunk(ck, _):
                base = pl.multiple_of(ck * CCH, CCH)
                pltpu.sync_copy(sorted_s.at[pl.ds(base, CCH)],
                                slist.at[w, pl.ds(base, CCH)])
                pltpu.sync_copy(sorted_d.at[pl.ds(base, CCH)],
                                dlist.at[w, pl.ds(base, CCH)])
                return 0

            lax.fori_loop(0, nck, flush_chunk, 0)

        flag = jnp.where(in_vmem, 1, 0)
        cnt_vec = jnp.full((16,), count, jnp.int32)
        cnt_v[...] = jnp.where(ii == 1, flag, cnt_vec)
        pltpu.sync_copy(cnt_v, counts.at[w])

    return k(src, dst)


def _sc_segmax(h, slist, dlist, counts, d):
    cg = 320 if d == 128 else 80

    @functools.partial(
        pl.kernel,
        mesh=_mesh(),
        compiler_params=_SC_PARAMS,
        out_type=jax.ShapeDtypeStruct((NPAD, d), jnp.float32),
        scratch_types=[
            pltpu.VMEM((NSEG, d), jnp.float32),
            pltpu.VMEM((cg, d), jnp.float32),
            pltpu.VMEM((cg, d), jnp.float32),
            pltpu.VMEM((cg,), jnp.int32),
            pltpu.VMEM((cg,), jnp.int32),
            pltpu.VMEM((cg,), jnp.int32),
            pltpu.VMEM((cg,), jnp.int32),
            pltpu.VMEM((16,), jnp.int32),
            pltpu.SemaphoreType.DMA,
            pltpu.SemaphoreType.DMA,
            pltpu.SemaphoreType.DMA,
            pltpu.SemaphoreType.DMA,
            pltpu.SemaphoreType.DMA,
            pltpu.SemaphoreType.DMA,
        ],
    )
    def k(h_hbm, slist_hbm, dlist_hbm, counts_hbm, agg, acc, rows0, rows1,
          idx0, idx1, dl0, dl1, cnt_v, ls0, ls1, ld0, ld1, g0, g1):
        rows = (rows0, rows1)
        idx = (idx0, idx1)
        dlb = (dl0, dl1)
        lssem = (ls0, ls1)
        ldsem = (ld0, ld1)
        gsem = (g0, g1)
        w = _wid()
        lo = w * NSEG
        pltpu.sync_copy(h_hbm.at[pl.ds(lo, NSEG)], acc)
        pltpu.sync_copy(counts_hbm.at[w], cnt_v)
        cvec = cnt_v[pl.ds(0, 16)]
        count = cvec[0]
        flag = cvec[1]
        nch = lax.div(count + (cg - 1), cg)
        R = d // 16

        def lists_cp(ck, b):
            base = pl.multiple_of(ck * cg, cg)
            return (pltpu.make_async_copy(slist_hbm.at[w, pl.ds(base, cg)],
                                          idx[b], lssem[b]),
                    pltpu.make_async_copy(dlist_hbm.at[w, pl.ds(base, cg)],
                                          dlb[b], ldsem[b]))

        def gather_cp(b):
            return pltpu.make_async_copy(h_hbm.at[idx[b]], rows[b], gsem[b])

        def prep_gather(ck, b):
            a1, a2 = lists_cp(ck, b)
            a1.wait()
            a2.wait()
            for j in range(cg // 16):
                v = idx[b][pl.ds(j * 16, 16)]
                idx[b][pl.ds(j * 16, 16)] = jnp.clip(v, 0, N - 1)
            gather_cp(b).start()

        @pl.when(nch > 0)
        def _prologue():
            a1, a2 = lists_cp(0, 0)
            a1.start()
            a2.start()

            @pl.when(nch > 1)
            def _():
                b1, b2 = lists_cp(1, 1)
                b1.start()
                b2.start()

            prep_gather(0, 0)

        def do_chunk(ck, b, carry):
            nb = 1 - b
            gather_cp(b).wait()

            @pl.when(ck + 1 < nch)
            def _():
                prep_gather(ck + 1, nb)

            base = pl.multiple_of(ck * cg, cg)
            m = jnp.minimum(cg, count - base)
            ng = lax.div(m, 16)

            def group_sorted(g, c):
                goff = pl.multiple_of(g * 16, 16)
                dlv = dlb[b][pl.ds(goff, 16)]
                cur = c[0]
                regs = list(c[1:])
                for e in range(16):
                    dl = dlv[e]

                    def switch(args, dl=dl):
                        oc = args[0]
                        for ci in range(R):
                            acc[oc, pl.ds(ci * 16, 16)] = args[1 + ci]
                        return (dl,) + tuple(acc[dl, pl.ds(ci * 16, 16)]
                                             for ci in range(R))

                    out = lax.cond(dl != cur, switch, lambda a: a,
                                   (cur, *regs))
                    cur = out[0]
                    regs = list(out[1:])
                    for ci in range(R):
                        sl = pl.ds(ci * 16, 16)
                        regs[ci] = jnp.maximum(regs[ci],
                                               rows[b][goff + e, sl])
                return (cur, *regs)

            def group_unsorted(g, c):
                goff = pl.multiple_of(g * 16, 16)
                dlv = dlb[b][pl.ds(goff, 16)]
                for e in range(16):
                    dl = dlv[e]
                    for ci in range(R):
                        sl = pl.ds(ci * 16, 16)
                        acc[dl, sl] = jnp.maximum(acc[dl, sl],
                                                  rows[b][goff + e, sl])
                return c

            carry = lax.cond(
                flag == 1,
                lambda c: lax.fori_loop(0, ng, group_sorted, c),
                lambda c: lax.fori_loop(0, ng, group_unsorted, c),
                carry)

            @pl.when(ck + 2 < nch)
            def _():
                a1, a2 = lists_cp(ck + 2, b)
                a1.start()
                a2.start()

            return carry

        def pair_body(kk, carry):
            for b in range(2):
                ck = kk * 2 + b
                carry = lax.cond(
                    ck < nch,
                    lambda c, ck=ck, b=b: do_chunk(ck, b, c),
                    lambda c: c, carry)
            return carry

        init = (jnp.int32(0),) + tuple(acc[0, pl.ds(ci * 16, 16)]
                                       for ci in range(R))
        fin = lax.fori_loop(0, lax.div(nch + 1, 2), pair_body, init)

        @pl.when(flag == 1)
        def _flush():
            for ci in range(R):
                acc[fin[0], pl.ds(ci * 16, 16)] = fin[1 + ci]

        pltpu.sync_copy(acc, agg.at[pl.ds(lo, NSEG)])

    return k(h, slist, dlist, counts)


def _sc_decoder(p, q, sidx, eidx):
    @functools.partial(
        pl.kernel,
        mesh=_mesh(),
        compiler_params=_SC_PARAMS,
        out_type=jax.ShapeDtypeStruct((PPAD,), jnp.float32),
        scratch_types=[
            pltpu.VMEM((NPAD,), jnp.float32),
            pltpu.VMEM((NPAD,), jnp.float32),
            pltpu.VMEM((PSEG,), jnp.int32),
            pltpu.VMEM((PSEG,), jnp.int32),
            pltpu.VMEM((PSEG,), jnp.float32),
        ],
    )
    def k(p_hbm, q_hbm, s_hbm, e_hbm, out, p_v, q_v, s_v, e_v, o_v):
        w = _wid()
        base = w * PSEG
        pltpu.sync_copy(p_hbm, p_v)
        pltpu.sync_copy(q_hbm, q_v)
        pltpu.sync_copy(s_hbm.at[pl.ds(base, PSEG)], s_v)
        pltpu.sync_copy(e_hbm.at[pl.ds(base, PSEG)], e_v)

        def body(j, _):
            off = pl.multiple_of(j * 16, 16)
            sv = s_v[pl.ds(off, 16)]
            ev = e_v[pl.ds(off, 16)]
            pv = plsc.load_gather(p_v, [sv])
            qv = plsc.load_gather(q_v, [ev])
            z = pv + qv
            o_v[pl.ds(off, 16)] = 1.0 / (1.0 + jnp.exp(-z))
            return 0

        lax.fori_loop(0, PSEG // 16, body, 0)
        pltpu.sync_copy(o_v, out.at[pl.ds(base, PSEG)])

    return k(p, q, sidx, eidx)


def _tc_lin(x, W0, b0):
    def body(x_ref, w_ref, b_ref, o_ref):
        o_ref[...] = jnp.dot(x_ref[...], w_ref[...],
                             preferred_element_type=jnp.float32) + b_ref[...]

    return pl.pallas_call(
        body,
        out_shape=jax.ShapeDtypeStruct((NPAD, W0.shape[1]), jnp.float32),
    )(x, W0, b0)


def _tc_layer(agg, h, Wl, bl, Wr, g, be):
    dout = Wl.shape[1]

    def body(a_ref, h_ref, wl_ref, bl_ref, wr_ref, g_ref, be_ref, o_ref):
        z = (jnp.dot(a_ref[...], wl_ref[...],
                     preferred_element_type=jnp.float32)
             + jnp.dot(h_ref[...], wr_ref[...],
                       preferred_element_type=jnp.float32)
             + bl_ref[...])
        zv = z[:N]
        mu = jnp.mean(zv, axis=0, keepdims=True)
        var = jnp.mean((zv - mu) ** 2, axis=0, keepdims=True)
        o = (z - mu) * (g_ref[...] * lax.rsqrt(var + 1e-5)) + be_ref[...]
        o_ref[...] = jnp.maximum(o, 0.0)

    return pl.pallas_call(
        body,
        out_shape=jax.ShapeDtypeStruct((NPAD, dout), jnp.float32),
    )(agg, h, Wl, bl, Wr, g, be)


def _tc_final(agg, h, Wl, bl, Wr, W2, bd):
    def body(a_ref, h_ref, wl_ref, bl_ref, wr_ref, w2_ref, bd_ref, p_ref,
             q_ref):
        z = (jnp.dot(a_ref[...], wl_ref[...],
                     preferred_element_type=jnp.float32)
             + jnp.dot(h_ref[...], wr_ref[...],
                       preferred_element_type=jnp.float32)
             + bl_ref[...])
        pq = jnp.dot(z, w2_ref[...], preferred_element_type=jnp.float32)
        p_ref[...] = pq[:, 0:1] + bd_ref[...]
        q_ref[...] = pq[:, 1:2]

    return pl.pallas_call(
        body,
        out_shape=(
            jax.ShapeDtypeStruct((NPAD, 1), jnp.float32),
            jax.ShapeDtypeStruct((NPAD, 1), jnp.float32),
        ),
    )(agg, h, Wl, bl, Wr, W2, bd)


def kernel(x, edge_index, junc_index_pair, W0, b0, Wl1, bl1, Wr1, g1, be1,
           Wl2, bl2, Wr2, g2, be2, Wl3, bl3, Wr3, Wd, bd):
    src = edge_index[0]
    dst = edge_index[1]
    sidx = jnp.pad(junc_index_pair[:, 0], (0, PPAD - NP))
    eidx = jnp.pad(junc_index_pair[:, 1], (0, PPAD - NP))
    x_pad = jnp.pad(x, ((0, NPAD - N), (0, 0)))
    W2 = jnp.concatenate([Wd[:Wd.shape[0] // 2], Wd[Wd.shape[0] // 2:]],
                         axis=1)

    slist, dlist, counts = _sc_compact(src, dst)
    h0 = _tc_lin(x_pad, W0, b0.reshape(1, -1))
    agg1 = _sc_segmax(h0, slist, dlist, counts, h0.shape[1])
    h1 = _tc_layer(agg1, h0, Wl1, bl1.reshape(1, -1), Wr1, g1.reshape(1, -1),
                   be1.reshape(1, -1))
    agg2 = _sc_segmax(h1, slist, dlist, counts, h1.shape[1])
    h2 = _tc_layer(agg2, h1, Wl2, bl2.reshape(1, -1), Wr2, g2.reshape(1, -1),
                   be2.reshape(1, -1))
    agg3 = _sc_segmax(h2, slist, dlist, counts, h2.shape[1])
    p, q = _tc_final(agg3, h2, Wl3, bl3.reshape(1, -1), Wr3, W2,
                     bd.reshape(1, 1))
    out = _sc_decoder(p.reshape(NPAD), q.reshape(NPAD), sidx, eidx)
    return out[:NP]
